# baseline (speedup 1.0000x reference)
import functools

import jax
import jax.numpy as jnp
from jax import lax
from jax.experimental import pallas as pl
from jax.experimental.pallas import tpu as pltpu
from jax.experimental.pallas import tpu_sc as plsc

B = 16384
EMB = 128
H1P = 128
H2P = 128

BIG_SIZES = (6041, 3439, 3884)
BIG_PADS = (6048, 3440, 3888)
BIG_W1 = (0, 4, 5)
BIG_ADJ = (-1, 0, -1)

SMALL_SIZES = (2, 7, 21, 81)
SMALL_OFFS = (0, 8, 16, 40)
SMALL_W1 = (1, 2, 3, 6)
SBLK = 128

NC = 2
NS = 16
NW = NC * NS
BPW = B // NW
CH = 64
NCH = BPW // CH
NSETS = 4


def _proj_body(ut, zt, it, gt, at_, ot, yt, w_ref, pu, pz, pi, ps_ref):
    for ref, out, wi in zip((ut, zt, it), (pu, pz, pi), BIG_W1):
        out[...] = jnp.dot(ref[...], w_ref[wi * EMB:(wi + 1) * EMB, :],
                           preferred_element_type=jnp.float32)
    @pl.when(pl.program_id(0) == 0)
    def _():
        ps_ref[...] = jnp.zeros((SBLK, H1P), jnp.float32)
        for ref, o, wi in zip((gt, at_, ot, yt), SMALL_OFFS, SMALL_W1):
            n = ref.shape[0]
            ps_ref[o:o + n, :] = jnp.dot(
                ref[...], w_ref[wi * EMB:(wi + 1) * EMB, :],
                preferred_element_type=jnp.float32)


def _small_body(g_ref, a_ref, o_ref_, y_ref, sb_ref, o_ref):
    rb = o_ref.shape[0]
    iota = lax.broadcasted_iota(jnp.int32, (rb, SBLK), 1)
    mh = jnp.zeros((rb, SBLK), jnp.float32)
    for ref, off in zip((g_ref, a_ref, o_ref_, y_ref), SMALL_OFFS):
        mh = mh + (ref[0, :].reshape(rb, 1) == iota - off).astype(jnp.float32)
    o_ref[...] = jnp.dot(mh, sb_ref[...], preferred_element_type=jnp.float32)


def _tail_body(h_ref, ss_ref, b1_ref, w2_ref, b2_ref, w3_ref, o_ref):
    x = jnp.maximum(h_ref[...] + ss_ref[...] + b1_ref[...], 0.0)
    h2 = jnp.maximum(
        jnp.dot(x, w2_ref[...], preferred_element_type=jnp.float32)
        + b2_ref[...], 0.0)
    y = jnp.dot(h2, w3_ref[...], preferred_element_type=jnp.float32)
    o_ref[...] = y[:, 0:1]


def _gather3_body(pu_hbm, pz_hbm, pi_hbm, uid_hbm, zid_hbm, iid_hbm,
                  out_hbm, idx_v,
                  b00, b01, b02, b10, b11, b12, b20, b21, b22, b30, b31, b32,
                  g0, g1, g2, g3, o0, o1, o2, o3, isem):
    wid = lax.axis_index("s") * NC + lax.axis_index("c")
    base = wid * BPW
    tabs = (pu_hbm, pz_hbm, pi_hbm)
    iw = []
    for f, ids in enumerate((uid_hbm, zid_hbm, iid_hbm)):
        iw.append(pltpu.async_copy(ids.at[pl.ds(base, BPW)],
                                   idx_v.at[pl.ds(f * BPW, BPW)], isem))
    for w in iw:
        w.wait()

    def _make_adjust(lo, hi):
        def _adjust(i, _):
            for f in (0, 2):
                sl = pl.ds(f * BPW + lo + i * 16, 16)
                idx_v[sl] = idx_v[sl] + BIG_ADJ[f]
            return 0
        return lax.fori_loop(0, (hi - lo) // 16, _adjust, 0)

    look = min(NSETS - 1, NCH)
    _make_adjust(0, look * CH)

    sets = ((b00, b01, b02, g0, o0), (b10, b11, b12, g1, o1),
            (b20, b21, b22, g2, o2), (b30, b31, b32, g3, o3))

    def fire(ch, setn):
        bufs = sets[setn]
        sem = bufs[3]
        return [pltpu.async_copy(
                    tabs[f].at[idx_v.at[pl.ds(f * BPW + ch * CH, CH)]],
                    bufs[f], sem)
                for f in range(3)]

    pend = {k: fire(k, k % NSETS) for k in range(look)}
    _make_adjust(look * CH, BPW)
    owait = [None] * NSETS
    for ch in range(NCH):
        nf = ch + look
        if nf < NCH:
            s = nf % NSETS
            if owait[s] is not None:
                owait[s].wait()
                owait[s] = None
            pend[nf] = fire(nf, s)
        for w in pend.pop(ch):
            w.wait()
        s = ch % NSETS
        b0, b1_, b2_ = sets[s][:3]

        def _sum_rows(r, _):
            for rr in range(2):
                row = r * 2 + rr
                for c in range(H1P // 16):
                    sl = pl.ds(c * 16, 16)
                    plsc.addupdate(b0.at[row, sl],
                                   b1_[row, sl] + b2_[row, sl])
            return 0

        lax.fori_loop(0, CH // 2, _sum_rows, 0)
        owait[s] = pltpu.async_copy(
            b0, out_hbm.at[pl.ds(base + ch * CH, CH)], sets[s][4])
    for s in range(NSETS):
        if owait[s] is not None:
            owait[s].wait()


@functools.cache
def _make_gather3():
    mesh = plsc.VectorSubcoreMesh(core_axis_name="c", subcore_axis_name="s",
                                  num_cores=NC, num_subcores=NS)
    buf = pltpu.VMEM((CH, H1P), jnp.float32)
    return functools.partial(
        pl.kernel,
        out_type=jax.ShapeDtypeStruct((B, H1P), jnp.float32),
        mesh=mesh,
        scratch_types=(
            [pltpu.VMEM((3 * BPW,), jnp.int32)]
            + [buf] * 12
            + [pltpu.SemaphoreType.DMA] * 9
        ),
    )(_gather3_body)


def kernel(user_id, gender, age, occup, zipc, item_id, year,
           user_table, gender_table, age_table, occup_table, zip_table,
           item_table, year_table, W1, b1, W2, b2, W3, b3):
    f32 = jnp.float32
    i32 = jnp.int32

    w1p = jnp.pad(W1, ((0, 0), (0, H1P - 100)))
    b1p = jnp.pad(b1, (0, H1P - 100)).reshape(1, H1P)
    w2p = jnp.pad(W2, ((0, H1P - 100), (0, H2P - 50)))
    b2p = jnp.zeros((H2P,), f32).at[:50].set(b2).at[50].set(1.0)
    b2p = b2p.reshape(1, H2P)
    w3p = jnp.zeros((H2P,), f32).at[:50].set(W3[:, 0]).at[50].set(b3[0])
    w3m = jnp.zeros((H2P, H2P), f32).at[:, 0].set(w3p)

    halves = tuple(p // 2 for p in BIG_PADS)

    p_user, p_zip, p_item, p_small = pl.pallas_call(
        _proj_body,
        grid=(2,),
        in_specs=[
            pl.BlockSpec((halves[0], EMB), lambda i: (i, 0)),
            pl.BlockSpec((halves[1], EMB), lambda i: (i, 0)),
            pl.BlockSpec((halves[2], EMB), lambda i: (i, 0)),
            pl.BlockSpec((SMALL_SIZES[0], EMB), lambda i: (0, 0)),
            pl.BlockSpec((SMALL_SIZES[1], EMB), lambda i: (0, 0)),
            pl.BlockSpec((SMALL_SIZES[2], EMB), lambda i: (0, 0)),
            pl.BlockSpec((SMALL_SIZES[3], EMB), lambda i: (0, 0)),
            pl.BlockSpec((7 * EMB, H1P), lambda i: (0, 0)),
        ],
        out_specs=(
            pl.BlockSpec((halves[0], H1P), lambda i: (i, 0)),
            pl.BlockSpec((halves[1], H1P), lambda i: (i, 0)),
            pl.BlockSpec((halves[2], H1P), lambda i: (i, 0)),
            pl.BlockSpec((SBLK, H1P), lambda i: (0, 0)),
        ),
        out_shape=(jax.ShapeDtypeStruct((BIG_PADS[0], H1P), f32),
                   jax.ShapeDtypeStruct((BIG_PADS[1], H1P), f32),
                   jax.ShapeDtypeStruct((BIG_PADS[2], H1P), f32),
                   jax.ShapeDtypeStruct((SBLK, H1P), f32)),
    )(user_table, zip_table, item_table, gender_table, age_table,
      occup_table, year_table, w1p)

    hpre = _make_gather3()(p_user, p_zip, p_item, user_id.astype(i32),
                           zipc.astype(i32), item_id.astype(i32))

    rb = 2048
    idspec = pl.BlockSpec((1, rb), lambda i: (0, i))
    ssum = pl.pallas_call(
        _small_body,
        grid=(B // rb,),
        in_specs=[
            idspec, idspec, idspec, idspec,
            pl.BlockSpec((SBLK, H1P), lambda i: (0, 0)),
        ],
        out_specs=pl.BlockSpec((rb, H1P), lambda i: (i, 0)),
        out_shape=jax.ShapeDtypeStruct((B, H1P), f32),
    )(gender.astype(i32).reshape(1, B), age.astype(i32).reshape(1, B),
      occup.astype(i32).reshape(1, B), year.astype(i32).reshape(1, B),
      p_small)

    rbt = 4096
    out = pl.pallas_call(
        _tail_body,
        grid=(B // rbt,),
        in_specs=[
            pl.BlockSpec((rbt, H1P), lambda i: (i, 0)),
            pl.BlockSpec((rbt, H1P), lambda i: (i, 0)),
            pl.BlockSpec((1, H1P), lambda i: (0, 0)),
            pl.BlockSpec((H1P, H2P), lambda i: (0, 0)),
            pl.BlockSpec((1, H2P), lambda i: (0, 0)),
            pl.BlockSpec((H2P, H2P), lambda i: (0, 0)),
        ],
        out_specs=pl.BlockSpec((rbt, 1), lambda i: (i, 0)),
        out_shape=jax.ShapeDtypeStruct((B, 1), f32),
    )(hpre, ssum, b1p, w2p, b2p, w3m)
    return out.reshape(B)


# speedup vs baseline: 1.0594x; 1.0594x over previous
import functools

import jax
import jax.numpy as jnp
from jax import lax
from jax.experimental import pallas as pl
from jax.experimental.pallas import tpu as pltpu
from jax.experimental.pallas import tpu_sc as plsc

B = 16384
EMB = 128
H1P = 128
H2P = 128

BIG_SIZES = (6041, 3439, 3884)
BIG_PADS = (6048, 3440, 3888)
BIG_W1 = (0, 4, 5)
BIG_ADJ = (-1, 0, -1)

SMALL_SIZES = (2, 7, 21, 81)
SMALL_OFFS = (0, 8, 16, 40)
SMALL_W1 = (1, 2, 3, 6)
SBLK = 128

NC = 2
NS = 16
NW = NC * NS
BPW = B // NW
CH = 64
NCH = BPW // CH
NSETS = 4


def _proj_body(ut, zt, it, gt, at_, ot, yt, w_ref, pu, pz, pi, ps_ref):
    for ref, out, wi in zip((ut, zt, it), (pu, pz, pi), BIG_W1):
        out[...] = jnp.dot(ref[...], w_ref[wi * EMB:(wi + 1) * EMB, :],
                           preferred_element_type=jnp.float32)
    @pl.when(pl.program_id(0) == 0)
    def _():
        ps_ref[...] = jnp.zeros((SBLK, H1P), jnp.float32)
        for ref, o, wi in zip((gt, at_, ot, yt), SMALL_OFFS, SMALL_W1):
            n = ref.shape[0]
            ps_ref[o:o + n, :] = jnp.dot(
                ref[...], w_ref[wi * EMB:(wi + 1) * EMB, :],
                preferred_element_type=jnp.float32)


def _small_body(g_ref, a_ref, o_ref_, y_ref, sb_ref, o_ref):
    rb = o_ref.shape[0]
    iota = lax.broadcasted_iota(jnp.int32, (rb, SBLK), 1)
    mh = jnp.zeros((rb, SBLK), jnp.float32)
    for ref, off in zip((g_ref, a_ref, o_ref_, y_ref), SMALL_OFFS):
        mh = mh + (ref[0, :].reshape(rb, 1) == iota - off).astype(jnp.float32)
    o_ref[...] = jnp.dot(mh, sb_ref[...], preferred_element_type=jnp.float32)


def _tail_body(h_ref, ss_ref, b1_ref, w2_ref, b2_ref, w3_ref, o_ref):
    x = jnp.maximum(h_ref[...] + ss_ref[...] + b1_ref[...], 0.0)
    h2 = jnp.maximum(
        jnp.dot(x, w2_ref[...], preferred_element_type=jnp.float32)
        + b2_ref[...], 0.0)
    o_ref[...] = jnp.sum(h2 * w3_ref[...], axis=1)


def _gather3_body(pu_hbm, pz_hbm, pi_hbm, uid_hbm, zid_hbm, iid_hbm,
                  out_hbm, idx_v,
                  b00, b01, b02, b10, b11, b12, b20, b21, b22, b30, b31, b32,
                  g0, g1, g2, g3, o0, o1, o2, o3, isem):
    wid = lax.axis_index("s") * NC + lax.axis_index("c")
    base = wid * BPW
    tabs = (pu_hbm, pz_hbm, pi_hbm)
    iw = []
    for f, ids in enumerate((uid_hbm, zid_hbm, iid_hbm)):
        iw.append(pltpu.async_copy(ids.at[pl.ds(base, BPW)],
                                   idx_v.at[pl.ds(f * BPW, BPW)], isem))
    for w in iw:
        w.wait()

    def _make_adjust(lo, hi):
        def _adjust(i, _):
            for f in (0, 2):
                sl = pl.ds(f * BPW + lo + i * 16, 16)
                idx_v[sl] = idx_v[sl] + BIG_ADJ[f]
            return 0
        return lax.fori_loop(0, (hi - lo) // 16, _adjust, 0)

    look = min(NSETS - 1, NCH)
    _make_adjust(0, look * CH)

    sets = ((b00, b01, b02, g0, o0), (b10, b11, b12, g1, o1),
            (b20, b21, b22, g2, o2), (b30, b31, b32, g3, o3))

    def fire(ch, setn):
        bufs = sets[setn]
        sem = bufs[3]
        return [pltpu.async_copy(
                    tabs[f].at[idx_v.at[pl.ds(f * BPW + ch * CH, CH)]],
                    bufs[f], sem)
                for f in range(3)]

    pend = {k: fire(k, k % NSETS) for k in range(look)}
    _make_adjust(look * CH, BPW)
    owait = [None] * NSETS
    for ch in range(NCH):
        nf = ch + look
        if nf < NCH:
            s = nf % NSETS
            if owait[s] is not None:
                owait[s].wait()
                owait[s] = None
            pend[nf] = fire(nf, s)
        for w in pend.pop(ch):
            w.wait()
        s = ch % NSETS
        b0, b1_, b2_ = sets[s][:3]

        def _sum_rows(r, _):
            for rr in range(2):
                row = r * 2 + rr
                for c in range(H1P // 16):
                    sl = pl.ds(c * 16, 16)
                    plsc.addupdate(b0.at[row, sl],
                                   b1_[row, sl] + b2_[row, sl])
            return 0

        lax.fori_loop(0, CH // 2, _sum_rows, 0)
        owait[s] = pltpu.async_copy(
            b0, out_hbm.at[pl.ds(base + ch * CH, CH)], sets[s][4])
    for s in range(NSETS):
        if owait[s] is not None:
            owait[s].wait()


@functools.cache
def _make_gather3():
    mesh = plsc.VectorSubcoreMesh(core_axis_name="c", subcore_axis_name="s",
                                  num_cores=NC, num_subcores=NS)
    buf = pltpu.VMEM((CH, H1P), jnp.float32)
    return functools.partial(
        pl.kernel,
        out_type=jax.ShapeDtypeStruct((B, H1P), jnp.float32),
        mesh=mesh,
        scratch_types=(
            [pltpu.VMEM((3 * BPW,), jnp.int32)]
            + [buf] * 12
            + [pltpu.SemaphoreType.DMA] * 9
        ),
    )(_gather3_body)


def kernel(user_id, gender, age, occup, zipc, item_id, year,
           user_table, gender_table, age_table, occup_table, zip_table,
           item_table, year_table, W1, b1, W2, b2, W3, b3):
    f32 = jnp.float32
    i32 = jnp.int32

    w1p = jnp.pad(W1, ((0, 0), (0, H1P - 100)))
    b1p = jnp.pad(b1, (0, H1P - 100)).reshape(1, H1P)
    w2p = jnp.pad(W2, ((0, H1P - 100), (0, H2P - 50)))
    b2p = jnp.zeros((H2P,), f32).at[:50].set(b2).at[50].set(1.0)
    b2p = b2p.reshape(1, H2P)
    w3p = jnp.zeros((H2P,), f32).at[:50].set(W3[:, 0]).at[50].set(b3[0])
    w3p = w3p.reshape(1, H2P)

    halves = tuple(p // 2 for p in BIG_PADS)

    p_user, p_zip, p_item, p_small = pl.pallas_call(
        _proj_body,
        grid=(2,),
        in_specs=[
            pl.BlockSpec((halves[0], EMB), lambda i: (i, 0)),
            pl.BlockSpec((halves[1], EMB), lambda i: (i, 0)),
            pl.BlockSpec((halves[2], EMB), lambda i: (i, 0)),
            pl.BlockSpec((SMALL_SIZES[0], EMB), lambda i: (0, 0)),
            pl.BlockSpec((SMALL_SIZES[1], EMB), lambda i: (0, 0)),
            pl.BlockSpec((SMALL_SIZES[2], EMB), lambda i: (0, 0)),
            pl.BlockSpec((SMALL_SIZES[3], EMB), lambda i: (0, 0)),
            pl.BlockSpec((7 * EMB, H1P), lambda i: (0, 0)),
        ],
        out_specs=(
            pl.BlockSpec((halves[0], H1P), lambda i: (i, 0)),
            pl.BlockSpec((halves[1], H1P), lambda i: (i, 0)),
            pl.BlockSpec((halves[2], H1P), lambda i: (i, 0)),
            pl.BlockSpec((SBLK, H1P), lambda i: (0, 0)),
        ),
        out_shape=(jax.ShapeDtypeStruct((BIG_PADS[0], H1P), f32),
                   jax.ShapeDtypeStruct((BIG_PADS[1], H1P), f32),
                   jax.ShapeDtypeStruct((BIG_PADS[2], H1P), f32),
                   jax.ShapeDtypeStruct((SBLK, H1P), f32)),
    )(user_table, zip_table, item_table, gender_table, age_table,
      occup_table, year_table, w1p)

    hpre = _make_gather3()(p_user, p_zip, p_item, user_id.astype(i32),
                           zipc.astype(i32), item_id.astype(i32))

    rb = 2048
    idspec = pl.BlockSpec((1, rb), lambda i: (0, i))
    ssum = pl.pallas_call(
        _small_body,
        grid=(B // rb,),
        in_specs=[
            idspec, idspec, idspec, idspec,
            pl.BlockSpec((SBLK, H1P), lambda i: (0, 0)),
        ],
        out_specs=pl.BlockSpec((rb, H1P), lambda i: (i, 0)),
        out_shape=jax.ShapeDtypeStruct((B, H1P), f32),
    )(gender.astype(i32).reshape(1, B), age.astype(i32).reshape(1, B),
      occup.astype(i32).reshape(1, B), year.astype(i32).reshape(1, B),
      p_small)

    rbt = 4096
    out = pl.pallas_call(
        _tail_body,
        grid=(B // rbt,),
        in_specs=[
            pl.BlockSpec((rbt, H1P), lambda i: (i, 0)),
            pl.BlockSpec((rbt, H1P), lambda i: (i, 0)),
            pl.BlockSpec((1, H1P), lambda i: (0, 0)),
            pl.BlockSpec((H1P, H2P), lambda i: (0, 0)),
            pl.BlockSpec((1, H2P), lambda i: (0, 0)),
            pl.BlockSpec((1, H2P), lambda i: (0, 0)),
        ],
        out_specs=pl.BlockSpec((rbt,), lambda i: (i,)),
        out_shape=jax.ShapeDtypeStruct((B,), f32),
    )(hpre, ssum, b1p, w2p, b2p, w3p)
    return out
